# baseline (speedup 1.0000x reference)
import functools
import math

import jax
import jax.numpy as jnp
from jax import lax
from jax.experimental import pallas as pl
from jax.experimental.pallas import tpu as pltpu
from jax.experimental.pallas import tpu_sc as plsc

D_EMBED = 64
SCALE = math.sqrt(D_EMBED)

NC = 2
NS = 16
NW = NC * NS

IDX_MINOR = 128
CHUNK = 1024
GATHERS = CHUNK // IDX_MINOR
LANES = 16


@functools.partial(jax.jit, static_argnums=())
def _embed_gather(idx2d, table):
    n_rows, _ = idx2d.shape
    B = n_rows * IDX_MINOR
    D = table.shape[1]
    b_per_w = B // NW
    n_chunks = b_per_w // CHUNK

    mesh = plsc.VectorSubcoreMesh(core_axis_name="c", subcore_axis_name="s")

    @functools.partial(
        pl.kernel,
        out_type=jax.ShapeDtypeStruct((B, D), jnp.float32),
        mesh=mesh,
        scratch_types=[
            pltpu.VMEM((GATHERS, IDX_MINOR), jnp.int32),
            pltpu.VMEM((CHUNK, D), jnp.float32),
            pltpu.SemaphoreType.DMA,
        ],
        compiler_params=pltpu.CompilerParams(use_tc_tiling_on_sc=False),
    )
    def body(idx_hbm, table_hbm, out_hbm, idx_v, rows_v, sem):
        wid = lax.axis_index("s") * NC + lax.axis_index("c")
        base = wid * b_per_w

        @pl.loop(0, n_chunks)
        def _chunk(c):
            off = base + c * CHUNK
            row_off = pl.multiple_of(off // IDX_MINOR, 8)
            pltpu.sync_copy(idx_hbm.at[pl.ds(row_off, GATHERS)], idx_v)
            cps = [
                pltpu.async_copy(
                    table_hbm.at[idx_v.at[g]],
                    rows_v.at[pl.ds(g * IDX_MINOR, IDX_MINOR)],
                    sem,
                )
                for g in range(GATHERS)
            ]
            for cp in cps:
                cp.wait()

            @pl.loop(0, CHUNK, unroll=8)
            def _scale(i):
                for j in range(D // LANES):
                    sl = pl.ds(j * LANES, LANES)
                    rows_v[i, sl] = rows_v[i, sl] * SCALE

            pltpu.sync_copy(rows_v, out_hbm.at[pl.ds(off, CHUNK)])

    return body(idx2d, table)


def kernel(x, table):
    B = x.shape[0] * x.shape[1]
    idx2d = x.reshape(B // IDX_MINOR, IDX_MINOR).astype(jnp.int32)
    out = _embed_gather(idx2d, table)
    return out.reshape(x.shape[0], x.shape[1], D_EMBED)
